# baseline (speedup 1.0000x reference)
import functools

import jax
import jax.numpy as jnp
import numpy as np
from jax.experimental import pallas as pl
from jax.experimental.pallas import tpu as pltpu

Z = 128
N1 = 131072
RW = 1024
C2 = 4096
KW1 = 26214
KW2 = 819

_MAX_IT = 160


def _mono(x):
    b = jax.lax.bitcast_convert_type(x, jnp.int32)
    return b ^ ((b >> 31) & jnp.int32(0x7FFFFFFF))


def _avg_floor(lo, hi):
    return (lo >> 1) + (hi >> 1) + (lo & hi & 1)


def kwinners_mask(m, col, k, ncols, sum_rows, max_rows, min_rows):
    one = jnp.int32(1)
    kf = jnp.float32(k)
    lo = min_rows(m) - one
    hi = max_rows(m)
    clo = jnp.full_like(lo, ncols)
    chi = jnp.zeros_like(lo)
    s = jnp.zeros_like(lo)
    jstar = jnp.full_like(lo, -1)
    need = jnp.zeros_like(lo)
    phase = jnp.zeros_like(lo)
    jlo = jnp.full_like(lo, -1)
    jhi = jnp.full_like(lo, ncols - 1)

    def cond(c):
        (it, phase, *_rest) = c
        return (it < _MAX_IT) & jnp.any(phase < 2)

    def body(c):
        (it, phase, lo, hi, clo, chi, s, jstar, need, jlo, jhi) = c
        p0 = phase == 0
        p1 = phase == 1

        span = hi.astype(jnp.float32) - lo.astype(jnp.float32)
        denom = jnp.maximum((clo - chi).astype(jnp.float32), 1.0)
        frac = (clo.astype(jnp.float32) - kf) / denom
        interp = lo + (span * frac).astype(jnp.int32)
        vprobe = jnp.where(it % 4 == 3, _avg_floor(lo, hi), interp)
        vprobe = jnp.clip(vprobe, lo + one, hi - one)
        jmid = (jlo + jhi) >> 1

        probe = jnp.where(p1, jmid, vprobe)
        pred = (p1 & (m == s) & (col <= probe)) | (~p1 & (m > probe))
        cnt = sum_rows(pred)

        sep = p0 & (cnt == k)
        ge = cnt >= k
        nlo = jnp.where(p0 & ge, probe, lo)
        nclo = jnp.where(p0 & ge, cnt, clo)
        nhi = jnp.where(p0 & ~ge, probe, hi)
        nchi = jnp.where(p0 & ~ge, cnt, chi)
        collapsed = p0 & ~sep & (nhi == nlo + one)
        ns = jnp.where(sep, probe, jnp.where(collapsed, nhi, s))
        nneed = jnp.where(collapsed, k - nchi, need)
        njstar = jnp.where(sep, -one, jstar)

        ge1 = p1 & (cnt >= need)
        njhi = jnp.where(ge1, probe, jhi)
        njlo = jnp.where(p1 & ~ge1, probe, jlo)
        done1 = p1 & (njhi == njlo + one)
        njstar = jnp.where(done1, njhi, njstar)

        nphase = jnp.where(sep | done1, 2, jnp.where(collapsed, 1, phase))
        return (it + one, nphase, nlo, nhi, nclo, nchi, ns, njstar, nneed,
                njlo, njhi)

    (_, _, _, _, _, _, s, jstar, _, _, _) = jax.lax.while_loop(
        cond, body,
        (jnp.int32(0), phase, lo, hi, clo, chi, s, jstar, need, jlo, jhi))
    return (m > s) | ((m == s) & (col <= jstar))


def _mv_kernel(x_ref, w_ref, b_ref, o_ref):
    acc = jax.lax.dot_general(
        x_ref[...], w_ref[...],
        dimension_numbers=(((1,), (1,)), ((), ())),
        preferred_element_type=jnp.float32)
    o_ref[...] = acc + b_ref[...]


def _fc1(x2, W1, b1w):
    BLK = 16384
    grid = N1 // BLK
    return pl.pallas_call(
        _mv_kernel,
        grid=(grid,),
        in_specs=[
            pl.BlockSpec((1, Z), lambda i: (0, 0)),
            pl.BlockSpec((BLK, Z), lambda i: (i, 0)),
            pl.BlockSpec((1, BLK), lambda i: (0, i)),
        ],
        out_specs=pl.BlockSpec((1, BLK), lambda i: (0, i)),
        out_shape=jax.ShapeDtypeStruct((1, N1), jnp.float32),
    )(x2, W1, b1w)


def _kw1_kernel(h_ref, o_ref):
    h = h_ref[...]
    m = _mono(h)
    r_iota = jax.lax.broadcasted_iota(jnp.int32, (8, 16384), 0)
    c_iota = jax.lax.broadcasted_iota(jnp.int32, (8, 16384), 1)
    lin = r_iota * 16384 + c_iota

    def sum_all(x):
        return jnp.sum(x.astype(jnp.int32))

    mask = kwinners_mask(m, lin, KW1, N1, sum_all, jnp.max, jnp.min)
    o_ref[...] = jnp.where(mask, h, 0.0)


def _kw1(h8):
    return pl.pallas_call(
        _kw1_kernel,
        out_shape=jax.ShapeDtypeStruct((8, 16384), jnp.float32),
    )(h8)


def _fc2_kernel(hm_ref, w2_ref, b2_ref, o_ref):
    g = jax.lax.dot_general(
        hm_ref[...], w2_ref[...],
        dimension_numbers=(((1,), (1,)), ((), ())),
        preferred_element_type=jnp.float32) + b2_ref[...]
    m = _mono(g)
    BR = g.shape[0]
    col = jax.lax.broadcasted_iota(jnp.int32, (BR, C2), 1)

    def sum_rows(x):
        return jnp.sum(x.astype(jnp.int32), axis=1, keepdims=True)

    def max_rows(x):
        return jnp.max(x, axis=1, keepdims=True)

    def min_rows(x):
        return jnp.min(x, axis=1, keepdims=True)

    mask = kwinners_mask(m, col, KW2, C2, sum_rows, max_rows, min_rows)
    o_ref[...] = jnp.where(mask, g, 0.0)


def _fc2(hm2d, W2, b2w):
    BR = 512
    grid = RW // BR
    return pl.pallas_call(
        _fc2_kernel,
        grid=(grid,),
        in_specs=[
            pl.BlockSpec((BR, Z), lambda i: (i, 0)),
            pl.BlockSpec((C2, Z), lambda i: (0, 0)),
            pl.BlockSpec((1, C2), lambda i: (0, 0)),
        ],
        out_specs=pl.BlockSpec((BR, C2), lambda i: (i, 0)),
        out_shape=jax.ShapeDtypeStruct((RW, C2), jnp.float32),
    )(hm2d, W2, b2w)


def kernel(x, W1, b1, W2, b2):
    x2 = x.reshape(1, Z)
    b1w = b1.reshape(1, N1)
    b2w = b2.reshape(1, C2)
    h = _fc1(x2, W1, b1w)
    hm = _kw1(h.reshape(8, 16384))
    y = _fc2(hm.reshape(RW, Z), W2, b2w)
    return y.reshape(C2, RW)


# speedup vs baseline: 2.4611x; 2.4611x over previous
import functools

import jax
import jax.numpy as jnp
import numpy as np
from jax.experimental import pallas as pl
from jax.experimental.pallas import tpu as pltpu

Z = 128
N1 = 131072
RW = 1024
C2 = 4096
KW1 = 26214
KW2 = 819

_MAX_IT = 160


def _mono(x):
    b = jax.lax.bitcast_convert_type(x, jnp.int32)
    return b ^ ((b >> 31) & jnp.int32(0x7FFFFFFF))


def _avg_floor(lo, hi):
    return (lo >> 1) + (hi >> 1) + (lo & hi & 1)


def _unmono_f(m):
    b = m ^ ((m >> 31) & jnp.int32(0x7FFFFFFF))
    return jax.lax.bitcast_convert_type(b, jnp.float32)


def kwinners_mask(m, col, k, ncols, sum_rows, max_rows, min_rows):
    one = jnp.int32(1)
    kf = jnp.float32(k)
    lo = min_rows(m) - one
    hi = max_rows(m)
    clo = jnp.full_like(lo, ncols)
    chi = jnp.zeros_like(lo)
    s0 = hi
    done0 = (lo + one == hi).astype(jnp.int32)

    def cond_a(c):
        (it, done, *_rest) = c
        return (it < _MAX_IT) & (jnp.min(done) == 0)

    def body_a(c):
        (it, done, lo, hi, clo, chi, s) = c
        vlo = _unmono_f(lo)
        vhi = _unmono_f(hi)
        denom = jnp.maximum((clo - chi).astype(jnp.float32), 1.0)
        frac = (clo.astype(jnp.float32) - kf) / denom
        t = vlo + (vhi - vlo) * frac
        interp = _mono(t)
        probe = jnp.where(it % 4 == 3, _avg_floor(lo, hi), interp)
        probe = jnp.clip(probe, lo + one, hi - one)
        cnt = sum_rows(m > probe)
        sep = cnt == k
        ge = cnt >= k
        nlo = jnp.where(ge, probe, lo)
        nclo = jnp.where(ge, cnt, clo)
        nhi = jnp.where(ge, hi, probe)
        nchi = jnp.where(ge, chi, cnt)
        collapsed = nhi == nlo + one
        dn = done != 0
        ndone = jnp.where(dn | sep | collapsed, one, done)
        ns = jnp.where(dn, s, jnp.where(sep, probe, nhi))
        nlo = jnp.where(dn, lo, nlo)
        nhi = jnp.where(dn, hi, nhi)
        nclo = jnp.where(dn, clo, nclo)
        nchi = jnp.where(dn, chi, nchi)
        return (it + one, ndone, nlo, nhi, nclo, nchi, ns)

    (_, _, lo, hi, clo, chi, s) = jax.lax.while_loop(
        cond_a, body_a,
        (jnp.int32(0), done0, lo, hi, clo, chi, s0))

    cgt = sum_rows(m > s)
    tie = cgt != k
    need = k - cgt
    eq = m == s
    cnteq = sum_rows(eq)

    jlo0 = jnp.full_like(lo, -1)
    jhi0 = jnp.full_like(lo, ncols - 1)
    act0 = (tie & (cnteq != need)).astype(jnp.int32)

    def cond_b(c):
        (act, _jlo, _jhi) = c
        return jnp.max(act) != 0

    def body_b(c):
        (act, jlo, jhi) = c
        a = act != 0
        jmid = (jlo + jhi) >> 1
        cnt = sum_rows(eq & (col <= jmid))
        ge = cnt >= need
        njhi = jnp.where(a & ge, jmid, jhi)
        njlo = jnp.where(a & ~ge, jmid, jlo)
        nact = jnp.where(a & (njhi != njlo + one), act, 0)
        return (nact, njlo, njhi)

    (_, _, jhi) = jax.lax.while_loop(cond_b, body_b, (act0, jlo0, jhi0))
    jstar = jnp.where(tie, jhi, -one)
    return (m > s) | (eq & (col <= jstar))


def _mv_kernel(x_ref, w_ref, b_ref, o_ref):
    acc = jax.lax.dot_general(
        x_ref[...], w_ref[...],
        dimension_numbers=(((1,), (1,)), ((), ())),
        preferred_element_type=jnp.float32)
    o_ref[...] = acc + b_ref[...]


def _fc1(x2, W1, b1w):
    BLK = 16384
    grid = N1 // BLK
    return pl.pallas_call(
        _mv_kernel,
        grid=(grid,),
        in_specs=[
            pl.BlockSpec((1, Z), lambda i: (0, 0)),
            pl.BlockSpec((BLK, Z), lambda i: (i, 0)),
            pl.BlockSpec((1, BLK), lambda i: (0, i)),
        ],
        out_specs=pl.BlockSpec((1, BLK), lambda i: (0, i)),
        out_shape=jax.ShapeDtypeStruct((1, N1), jnp.float32),
    )(x2, W1, b1w)


def _kw1_kernel(h_ref, o_ref):
    h = h_ref[...]
    m = _mono(h)
    r_iota = jax.lax.broadcasted_iota(jnp.int32, (8, 16384), 0)
    c_iota = jax.lax.broadcasted_iota(jnp.int32, (8, 16384), 1)
    lin = r_iota * 16384 + c_iota

    def sum_all(x):
        return jnp.sum(x.astype(jnp.int32))

    mask = kwinners_mask(m, lin, KW1, N1, sum_all, jnp.max, jnp.min)
    o_ref[...] = jnp.where(mask, h, 0.0)


def _kw1(h8):
    return pl.pallas_call(
        _kw1_kernel,
        out_shape=jax.ShapeDtypeStruct((8, 16384), jnp.float32),
    )(h8)


def _fc2_kernel(hm_ref, w2_ref, b2_ref, o_ref):
    g = jax.lax.dot_general(
        hm_ref[...], w2_ref[...],
        dimension_numbers=(((1,), (1,)), ((), ())),
        preferred_element_type=jnp.float32) + b2_ref[...]
    m = _mono(g)
    BR = g.shape[0]
    col = jax.lax.broadcasted_iota(jnp.int32, (BR, C2), 1)

    def sum_rows(x):
        return jnp.sum(x.astype(jnp.int32), axis=1, keepdims=True)

    def max_rows(x):
        return jnp.max(x, axis=1, keepdims=True)

    def min_rows(x):
        return jnp.min(x, axis=1, keepdims=True)

    mask = kwinners_mask(m, col, KW2, C2, sum_rows, max_rows, min_rows)
    o_ref[...] = jnp.where(mask, g, 0.0)


def _fc2(hm2d, W2, b2w):
    BR = 512
    grid = RW // BR
    return pl.pallas_call(
        _fc2_kernel,
        grid=(grid,),
        in_specs=[
            pl.BlockSpec((BR, Z), lambda i: (i, 0)),
            pl.BlockSpec((C2, Z), lambda i: (0, 0)),
            pl.BlockSpec((1, C2), lambda i: (0, 0)),
        ],
        out_specs=pl.BlockSpec((BR, C2), lambda i: (i, 0)),
        out_shape=jax.ShapeDtypeStruct((RW, C2), jnp.float32),
    )(hm2d, W2, b2w)


def kernel(x, W1, b1, W2, b2):
    x2 = x.reshape(1, Z)
    b1w = b1.reshape(1, N1)
    b2w = b2.reshape(1, C2)
    h = _fc1(x2, W1, b1w)
    hm = _kw1(h.reshape(8, 16384))
    y = _fc2(hm.reshape(RW, Z), W2, b2w)
    return y.reshape(C2, RW)
